# baseline (speedup 1.0000x reference)
import functools

import jax
import jax.numpy as jnp
from jax import lax
from jax.experimental import pallas as pl
from jax.experimental.pallas import tpu as pltpu
from jax.experimental.pallas import tpu_sc as plsc

K_TAPS = 4
DILATION = 2
BC = 128


def _sc_copy_plane(x1, B, C, T):
    R = B * C
    info = plsc.get_sparse_core_info()
    num_workers = info.num_cores * info.num_subcores
    rows_per_w = R // num_workers
    mesh = plsc.VectorSubcoreMesh(core_axis_name="c", subcore_axis_name="s")

    @functools.partial(
        pl.kernel,
        mesh=mesh,
        out_type=jax.ShapeDtypeStruct((B * K_TAPS * C * T,), jnp.float32),
        scratch_types=[pltpu.SemaphoreType.DMA],
    )
    def copy_plane(x_hbm, out_hbm, sem):
        cid = lax.axis_index("c")
        sid = lax.axis_index("s")
        wid = sid * info.num_cores + cid
        row0 = wid * rows_per_w
        b = row0 // C
        c0 = row0 - b * C
        dst = (b * K_TAPS + (K_TAPS - 1)) * C + c0
        pltpu.async_copy(
            x_hbm.at[pl.ds(pl.multiple_of(row0 * T, 8), rows_per_w * T)],
            out_hbm.at[pl.ds(pl.multiple_of(dst * T, 8), rows_per_w * T)],
            sem,
        ).wait()

    return copy_plane(x1)


def kernel(x):
    B, C, T = x.shape
    o4 = _sc_copy_plane(x.reshape(B * C * T), B, C, T).reshape(B, K_TAPS, C, T)

    def body(x_ref, o_alias_ref, o_ref):
        xv = x_ref[0]
        for k in range(K_TAPS - 1):
            s = (K_TAPS - 1 - k) * DILATION
            o_ref[0, k] = jnp.concatenate(
                [jnp.zeros((BC, s), jnp.float32), xv[:, : T - s]], axis=1
            )

    out4 = pl.pallas_call(
        body,
        grid=(B, C // BC),
        in_specs=[
            pl.BlockSpec((1, BC, T), lambda b, c: (b, c, 0)),
            pl.BlockSpec(memory_space=pl.ANY),
        ],
        out_specs=pl.BlockSpec((1, K_TAPS - 1, BC, T), lambda b, c: (b, 0, c, 0)),
        out_shape=jax.ShapeDtypeStruct((B, K_TAPS, C, T), jnp.float32),
        input_output_aliases={1: 0},
    )(x, o4)
    return out4.reshape(B, K_TAPS * C, T)


# speedup vs baseline: 22.8693x; 22.8693x over previous
import jax
import jax.numpy as jnp
from jax.experimental import pallas as pl

K_TAPS = 4
DILATION = 2
BC = 256


def kernel(x):
    B, C, T = x.shape

    def body(x_ref, o_ref):
        xv = x_ref[0]
        for k in range(K_TAPS):
            s = (K_TAPS - 1 - k) * DILATION
            if s == 0:
                o_ref[0, k] = xv
            else:
                o_ref[0, k] = jnp.concatenate(
                    [jnp.zeros((BC, s), jnp.float32), xv[:, : T - s]], axis=1
                )

    out4 = pl.pallas_call(
        body,
        grid=(B, C // BC),
        in_specs=[pl.BlockSpec((1, BC, T), lambda b, c: (b, c, 0))],
        out_specs=pl.BlockSpec((1, K_TAPS, BC, T), lambda b, c: (b, 0, c, 0)),
        out_shape=jax.ShapeDtypeStruct((B, K_TAPS, C, T), jnp.float32),
    )(x)
    return out4.reshape(B, K_TAPS * C, T)


# speedup vs baseline: 22.9814x; 1.0049x over previous
import jax
import jax.numpy as jnp
from jax.experimental import pallas as pl

K_TAPS = 4
DILATION = 2
BC = 192


def kernel(x):
    B, C, T = x.shape

    def body(x_ref, o_ref):
        xv = x_ref[0]
        for k in range(K_TAPS):
            s = (K_TAPS - 1 - k) * DILATION
            if s == 0:
                o_ref[0, k] = xv
            else:
                o_ref[0, k] = jnp.concatenate(
                    [jnp.zeros((BC, s), jnp.float32), xv[:, : T - s]], axis=1
                )

    out4 = pl.pallas_call(
        body,
        grid=(B, C // BC),
        in_specs=[pl.BlockSpec((1, BC, T), lambda b, c: (b, c, 0))],
        out_specs=pl.BlockSpec((1, K_TAPS, BC, T), lambda b, c: (b, 0, c, 0)),
        out_shape=jax.ShapeDtypeStruct((B, K_TAPS, C, T), jnp.float32),
    )(x)
    return out4.reshape(B, K_TAPS * C, T)
